# baseline (speedup 1.0000x reference)
import jax
import jax.numpy as jnp
from jax.experimental import pallas as pl

_N = 1_000_000
_D = 64
_BLOCK = 20_000


def _copy_body(x_ref, o_ref):
    o_ref[...] = x_ref[...]

    @pl.when(pl.program_id(0) == 0)
    def _():
        two = jnp.full((1, _D), 2.0, jnp.float32)
        o_ref[pl.ds(1, 1), :] = two
        o_ref[pl.ds(3, 1), :] = two


def kernel(x):
    return pl.pallas_call(
        _copy_body,
        grid=(_N // _BLOCK,),
        in_specs=[pl.BlockSpec((_BLOCK, _D), lambda i: (i, 0))],
        out_specs=pl.BlockSpec((_BLOCK, _D), lambda i: (i, 0)),
        out_shape=jax.ShapeDtypeStruct((_N, _D), jnp.float32),
    )(x)
